# baseline (speedup 1.0000x reference)
import functools

import jax
import jax.numpy as jnp
from jax import lax
from jax.experimental import pallas as pl
from jax.experimental.pallas import tpu as pltpu
from jax.experimental.pallas import tpu_sc as plsc

N = 10000
E = 320000
D = 128
NG = 64
BN_EPS = 128.0

NC = 2
NS = 16
NW = NC * NS
EPW = E // NW
CHUNK = 40
NB = 5
NGRP = EPW // (NB * CHUNK)
NP = 10240
RPT = NP // NS


def _agg_body(h_hbm, idx_hbm, zero_hbm, out_hbm,
              sidx0, didx0, sidx1, didx1, rows, shared, gsem, ssem, isem):
    c = lax.axis_index("c")
    s = lax.axis_index("s")
    wid = c * NS + s

    pltpu.async_copy(zero_hbm, shared.at[pl.ds(s * RPT, RPT)], isem)
    pltpu.sync_copy(idx_hbm.at[0, wid, 0], sidx0)
    pltpu.sync_copy(idx_hbm.at[1, wid, 0], didx0)

    def scatter_wait(b):
        pltpu.make_async_copy(rows.at[b], shared.at[didx0.at[b]],
                              ssem).wait()

    def do_group(sidx, drain_prev):
        gathers = []
        for b in range(NB):
            if drain_prev is None:
                scatter_wait(b)
            elif drain_prev:
                @pl.when(drain_prev())
                def _(b=b):
                    scatter_wait(b)
            gathers.append(
                pltpu.async_copy(h_hbm.at[sidx.at[b]], rows.at[b], gsem))
        return gathers

    def issue_scatters(didx, gathers):
        for b in range(NB):
            gathers[b].wait()
            pltpu.async_copy(rows.at[b], shared.at[didx.at[b]],
                             ssem, add=True)

    K = NGRP // 2

    def body(k, carry):
        @pl.when(k > 0)
        def _():
            pltpu.make_async_copy(idx_hbm.at[0, wid, 0], sidx0, isem).wait()
            pltpu.make_async_copy(idx_hbm.at[1, wid, 0], didx0, isem).wait()
        i1a = pltpu.async_copy(idx_hbm.at[0, wid, 2 * k + 1], sidx1, isem)
        i1b = pltpu.async_copy(idx_hbm.at[1, wid, 2 * k + 1], didx1, isem)
        g = do_group(sidx0, (lambda: k > 0))
        @pl.when(k == 0)
        def _():
            pltpu.make_async_copy(zero_hbm, shared.at[pl.ds(s * RPT, RPT)],
                                  isem).wait()
            plsc.subcore_barrier()
        issue_scatters(didx0, g)
        i1a.wait()
        i1b.wait()
        g = do_group(sidx1, None)
        @pl.when(k < K - 1)
        def _():
            pltpu.async_copy(idx_hbm.at[0, wid, 2 * k + 2], sidx0, isem)
            pltpu.async_copy(idx_hbm.at[1, wid, 2 * k + 2], didx0, isem)
        issue_scatters(didx1, g)
        return carry

    lax.fori_loop(0, K, body, 0)
    for b in range(NB):
        scatter_wait(b)
    plsc.subcore_barrier()

    pltpu.sync_copy(shared.at[pl.ds(s * RPT, RPT)],
                    out_hbm.at[pl.ds(c * NP + s * RPT, RPT)])


_agg = functools.partial(
    pl.kernel,
    mesh=plsc.VectorSubcoreMesh(core_axis_name="c", subcore_axis_name="s"),
    out_type=jax.ShapeDtypeStruct((2 * NP, D), jnp.float32),
    scratch_types=[
        pltpu.VMEM((NB, CHUNK), jnp.int32),
        pltpu.VMEM((NB, CHUNK), jnp.int32),
        pltpu.VMEM((NB, CHUNK), jnp.int32),
        pltpu.VMEM((NB, CHUNK), jnp.int32),
        pltpu.VMEM((NB, CHUNK, D), jnp.float32),
        pltpu.VMEM_SHARED((NP, D), jnp.float32),
        pltpu.SemaphoreType.DMA,
        pltpu.SemaphoreType.DMA,
        pltpu.SemaphoreType.DMA,
    ],
)(_agg_body)


def _dense_body(h_ref, a_ref, eps_ref, w_ref, b_ref, g_ref, be_ref, out_ref):
    agg = a_ref[0:N, :] + a_ref[NP:NP + N, :]
    z0 = (1.0 + eps_ref[...]) * h_ref[...] + agg
    z = jnp.dot(z0, w_ref[...], preferred_element_type=jnp.float32) + b_ref[...]
    m = jnp.mean(z, axis=0, keepdims=True)
    v = jnp.mean(z * z, axis=0, keepdims=True) - m * m
    zn = (z - m) * lax.rsqrt(v + BN_EPS) * g_ref[...] + be_ref[...]
    out_ref[...] = jnp.where(zn >= 0, zn, 1e-4 * zn)


def _dense(h, agg2, eps, w, b, g, be):
    return pl.pallas_call(
        _dense_body,
        out_shape=jax.ShapeDtypeStruct((N, D), jnp.float32),
        compiler_params=pltpu.CompilerParams(
            vmem_limit_bytes=100 * 1024 * 1024),
    )(h, agg2, eps.reshape(1, 1), w, b.reshape(1, D), g.reshape(1, D),
      be.reshape(1, D))


def _head_body(h1, h2, h3, h4in, a4, eps4, w4d, b4d, g4, be4,
               bt_c, wk1, bk1,
               wa, ba, wb, bb, wf, bf, out_ref):
    agg = a4[0:N, :] + a4[NP:NP + N, :]
    z0 = (1.0 + eps4[...]) * h4in[...] + agg
    z = jnp.dot(z0, w4d[...], preferred_element_type=jnp.float32) + b4d[...]
    m = jnp.mean(z, axis=0, keepdims=True)
    v = jnp.mean(z * z, axis=0, keepdims=True) - m * m
    zn = (z - m) * lax.rsqrt(v + BN_EPS) * g4[...] + be4[...]
    h4 = jnp.where(zn >= 0, zn, 1e-4 * zn)

    oh = (bt_c[...] == lax.broadcasted_iota(jnp.int32, (N, NG), 1)
          ).astype(jnp.float32)
    pool = lax.dot_general(oh, h4, (((0,), (0,)), ((), ())),
                           preferred_element_type=jnp.float32)
    hp = jnp.dot(oh, pool, preferred_element_type=jnp.float32)
    s = (jnp.dot(h1[...], wk1[0 * D:1 * D], preferred_element_type=jnp.float32)
         + jnp.dot(h2[...], wk1[1 * D:2 * D], preferred_element_type=jnp.float32)
         + jnp.dot(h3[...], wk1[2 * D:3 * D], preferred_element_type=jnp.float32)
         + jnp.dot(h4, wk1[3 * D:4 * D], preferred_element_type=jnp.float32)
         + jnp.dot(hp, wk1[4 * D:5 * D], preferred_element_type=jnp.float32)
         + bk1[...])
    s = jnp.dot(s, wa[...], preferred_element_type=jnp.float32) + ba[...]
    s = jnp.where(s >= 0, s, 0.01 * s)
    s = jnp.dot(s, wb[...], preferred_element_type=jnp.float32) + bb[...]
    s = jnp.where(s >= 0, s, 0.01 * s)
    o = jnp.dot(s, wf[...], preferred_element_type=jnp.float32) + bf[...]
    out_ref[...] = 1.0 / (1.0 + jnp.exp(-o))


def _head(h1, h2, h3, h4in, a4, eps4, w4d, b4d, g4, be4,
          batch, Wk1, bk1, Wk, bk, Wf, bf):
    return pl.pallas_call(
        _head_body,
        out_shape=jax.ShapeDtypeStruct((N, 1), jnp.float32),
        compiler_params=pltpu.CompilerParams(
            vmem_limit_bytes=100 * 1024 * 1024),
    )(h1, h2, h3, h4in, a4, eps4.reshape(1, 1), w4d, b4d.reshape(1, D),
      g4.reshape(1, D), be4.reshape(1, D),
      batch.reshape(N, 1),
      Wk1, bk1.reshape(1, -1),
      Wk[0], bk[0].reshape(1, -1), Wk[1], bk[1].reshape(1, -1),
      Wf, bf.reshape(1, -1))


def kernel(x, edge_index, batch, W1, b1, g1, be1, eps1, Wc, bc, gc, bec,
           epsc, Wk1, bk1, Wk, bk, Wf, bf):
    idx5 = edge_index.reshape(2, NW, NGRP, NB, CHUNK)
    zero = jnp.zeros((RPT, D), jnp.float32)

    h = x
    hs = []
    layer_params = [(eps1, W1, b1, g1, be1)] + [
        (epsc[i], Wc[i], bc[i], gc[i], bec[i]) for i in range(3)]
    for li, (eps, w, b, g, be) in enumerate(layer_params):
        agg2 = _agg(h, idx5, zero)
        if li < 3:
            h = _dense(h, agg2, eps, w, b, g, be)
            hs.append(h)
        else:
            return _head(hs[0], hs[1], hs[2], h, agg2, eps, w, b, g, be,
                         batch, Wk1, bk1, Wk, bk, Wf, bf)


# speedup vs baseline: 1.0296x; 1.0296x over previous
import functools

import jax
import jax.numpy as jnp
from jax import lax
from jax.experimental import pallas as pl
from jax.experimental.pallas import tpu as pltpu
from jax.experimental.pallas import tpu_sc as plsc

N = 10000
E = 320000
D = 128
NG = 64
BN_EPS = 128.0

NC = 2
NS = 16
NW = NC * NS
EPW = E // NW
CHUNK = 50
NB = 5
NGRP = EPW // (NB * CHUNK)
NP = 10240
RPT = NP // NS


def _agg_body(h_hbm, idx_hbm, zero_hbm, out_hbm,
              sidx0, didx0, sidx1, didx1, rows, shared, gsem, ssem, isem):
    c = lax.axis_index("c")
    s = lax.axis_index("s")
    wid = c * NS + s

    pltpu.async_copy(zero_hbm, shared.at[pl.ds(s * RPT, RPT)], isem)
    pltpu.sync_copy(idx_hbm.at[0, wid, 0], sidx0)
    pltpu.sync_copy(idx_hbm.at[1, wid, 0], didx0)

    def scatter_wait(b):
        pltpu.make_async_copy(rows.at[b], shared.at[didx0.at[b]],
                              ssem).wait()

    def do_group(sidx, drain_prev):
        gathers = []
        for b in range(NB):
            if drain_prev is None:
                scatter_wait(b)
            elif drain_prev:
                @pl.when(drain_prev())
                def _(b=b):
                    scatter_wait(b)
            gathers.append(
                pltpu.async_copy(h_hbm.at[sidx.at[b]], rows.at[b], gsem))
        return gathers

    def issue_scatters(didx, gathers):
        for b in range(NB):
            gathers[b].wait()
            pltpu.async_copy(rows.at[b], shared.at[didx.at[b]],
                             ssem, add=True)

    K = NGRP // 2

    def body(k, carry):
        @pl.when(k > 0)
        def _():
            pltpu.make_async_copy(idx_hbm.at[0, wid, 0], sidx0, isem).wait()
            pltpu.make_async_copy(idx_hbm.at[1, wid, 0], didx0, isem).wait()
        i1a = pltpu.async_copy(idx_hbm.at[0, wid, 2 * k + 1], sidx1, isem)
        i1b = pltpu.async_copy(idx_hbm.at[1, wid, 2 * k + 1], didx1, isem)
        g = do_group(sidx0, (lambda: k > 0))
        @pl.when(k == 0)
        def _():
            pltpu.make_async_copy(zero_hbm, shared.at[pl.ds(s * RPT, RPT)],
                                  isem).wait()
            plsc.subcore_barrier()
        issue_scatters(didx0, g)
        i1a.wait()
        i1b.wait()
        g = do_group(sidx1, None)
        @pl.when(k < K - 1)
        def _():
            pltpu.async_copy(idx_hbm.at[0, wid, 2 * k + 2], sidx0, isem)
            pltpu.async_copy(idx_hbm.at[1, wid, 2 * k + 2], didx0, isem)
        issue_scatters(didx1, g)
        return carry

    lax.fori_loop(0, K, body, 0)
    for b in range(NB):
        scatter_wait(b)
    plsc.subcore_barrier()

    pltpu.sync_copy(shared.at[pl.ds(s * RPT, RPT)],
                    out_hbm.at[pl.ds(c * NP + s * RPT, RPT)])


_agg = functools.partial(
    pl.kernel,
    mesh=plsc.VectorSubcoreMesh(core_axis_name="c", subcore_axis_name="s"),
    out_type=jax.ShapeDtypeStruct((2 * NP, D), jnp.float32),
    scratch_types=[
        pltpu.VMEM((NB, CHUNK), jnp.int32),
        pltpu.VMEM((NB, CHUNK), jnp.int32),
        pltpu.VMEM((NB, CHUNK), jnp.int32),
        pltpu.VMEM((NB, CHUNK), jnp.int32),
        pltpu.VMEM((NB, CHUNK, D), jnp.float32),
        pltpu.VMEM_SHARED((NP, D), jnp.float32),
        pltpu.SemaphoreType.DMA,
        pltpu.SemaphoreType.DMA,
        pltpu.SemaphoreType.DMA,
    ],
)(_agg_body)


def _dense_body(h_ref, a_ref, eps_ref, w_ref, b_ref, g_ref, be_ref, out_ref):
    agg = a_ref[0:N, :] + a_ref[NP:NP + N, :]
    z0 = (1.0 + eps_ref[...]) * h_ref[...] + agg
    z = jnp.dot(z0, w_ref[...], preferred_element_type=jnp.float32) + b_ref[...]
    m = jnp.mean(z, axis=0, keepdims=True)
    v = jnp.mean(z * z, axis=0, keepdims=True) - m * m
    zn = (z - m) * lax.rsqrt(v + BN_EPS) * g_ref[...] + be_ref[...]
    out_ref[...] = jnp.where(zn >= 0, zn, 1e-4 * zn)


def _dense(h, agg2, eps, w, b, g, be):
    return pl.pallas_call(
        _dense_body,
        out_shape=jax.ShapeDtypeStruct((N, D), jnp.float32),
        compiler_params=pltpu.CompilerParams(
            vmem_limit_bytes=100 * 1024 * 1024),
    )(h, agg2, eps.reshape(1, 1), w, b.reshape(1, D), g.reshape(1, D),
      be.reshape(1, D))


def _head_body(h1, h2, h3, h4in, a4, eps4, w4d, b4d, g4, be4,
               bt_c, wk1, bk1,
               wa, ba, wb, bb, wf, bf, out_ref):
    agg = a4[0:N, :] + a4[NP:NP + N, :]
    z0 = (1.0 + eps4[...]) * h4in[...] + agg
    z = jnp.dot(z0, w4d[...], preferred_element_type=jnp.float32) + b4d[...]
    m = jnp.mean(z, axis=0, keepdims=True)
    v = jnp.mean(z * z, axis=0, keepdims=True) - m * m
    zn = (z - m) * lax.rsqrt(v + BN_EPS) * g4[...] + be4[...]
    h4 = jnp.where(zn >= 0, zn, 1e-4 * zn)

    oh = (bt_c[...] == lax.broadcasted_iota(jnp.int32, (N, NG), 1)
          ).astype(jnp.float32)
    pool = lax.dot_general(oh, h4, (((0,), (0,)), ((), ())),
                           preferred_element_type=jnp.float32)
    hp = jnp.dot(oh, pool, preferred_element_type=jnp.float32)
    s = (jnp.dot(h1[...], wk1[0 * D:1 * D], preferred_element_type=jnp.float32)
         + jnp.dot(h2[...], wk1[1 * D:2 * D], preferred_element_type=jnp.float32)
         + jnp.dot(h3[...], wk1[2 * D:3 * D], preferred_element_type=jnp.float32)
         + jnp.dot(h4, wk1[3 * D:4 * D], preferred_element_type=jnp.float32)
         + jnp.dot(hp, wk1[4 * D:5 * D], preferred_element_type=jnp.float32)
         + bk1[...])
    s = jnp.dot(s, wa[...], preferred_element_type=jnp.float32) + ba[...]
    s = jnp.where(s >= 0, s, 0.01 * s)
    s = jnp.dot(s, wb[...], preferred_element_type=jnp.float32) + bb[...]
    s = jnp.where(s >= 0, s, 0.01 * s)
    o = jnp.dot(s, wf[...], preferred_element_type=jnp.float32) + bf[...]
    out_ref[...] = 1.0 / (1.0 + jnp.exp(-o))


def _head(h1, h2, h3, h4in, a4, eps4, w4d, b4d, g4, be4,
          batch, Wk1, bk1, Wk, bk, Wf, bf):
    return pl.pallas_call(
        _head_body,
        out_shape=jax.ShapeDtypeStruct((N, 1), jnp.float32),
        compiler_params=pltpu.CompilerParams(
            vmem_limit_bytes=100 * 1024 * 1024),
    )(h1, h2, h3, h4in, a4, eps4.reshape(1, 1), w4d, b4d.reshape(1, D),
      g4.reshape(1, D), be4.reshape(1, D),
      batch.reshape(N, 1),
      Wk1, bk1.reshape(1, -1),
      Wk[0], bk[0].reshape(1, -1), Wk[1], bk[1].reshape(1, -1),
      Wf, bf.reshape(1, -1))


def kernel(x, edge_index, batch, W1, b1, g1, be1, eps1, Wc, bc, gc, bec,
           epsc, Wk1, bk1, Wk, bk, Wf, bf):
    idx5 = edge_index.reshape(2, NW, NGRP, NB, CHUNK)
    zero = jnp.zeros((RPT, D), jnp.float32)

    h = x
    hs = []
    layer_params = [(eps1, W1, b1, g1, be1)] + [
        (epsc[i], Wc[i], bc[i], gc[i], bec[i]) for i in range(3)]
    for li, (eps, w, b, g, be) in enumerate(layer_params):
        agg2 = _agg(h, idx5, zero)
        if li < 3:
            h = _dense(h, agg2, eps, w, b, g, be)
            hs.append(h)
        else:
            return _head(hs[0], hs[1], hs[2], h, agg2, eps, w, b, g, be,
                         batch, Wk1, bk1, Wk, bk, Wf, bf)
